# baseline (speedup 1.0000x reference)
import functools

import jax
import jax.numpy as jnp
from jax import lax
from jax.experimental import pallas as pl
from jax.experimental.pallas import tpu as pltpu
from jax.experimental.pallas import tpu_sc as plsc

N = 10000
D = 128
E = 320000
NT = 10240
NC = 2
NS = 16
NW = NC * NS
B = 128
CH = 80
HCH = CH // 2
EP = NW * CH * B
ROWS_PER_TILE = NT // NS


def _mesh():
    return plsc.VectorSubcoreMesh(
        core_axis_name="c", subcore_axis_name="s", num_cores=NC, num_subcores=NS
    )


@functools.lru_cache(maxsize=None)
def _build_hist():
    @functools.partial(
        pl.kernel,
        mesh=_mesh(),
        out_type=jax.ShapeDtypeStruct((NC * NT,), jnp.float32),
        scratch_types=[
            pltpu.VMEM((CH, B), jnp.int32),
            pltpu.VMEM((B,), jnp.float32),
            pltpu.VMEM((ROWS_PER_TILE,), jnp.float32),
            pltpu.VMEM_SHARED((NT,), jnp.float32),
            pltpu.SemaphoreType.DMA,
        ],
    )
    def hist(dst_hbm, out_hbm, idx_v, ones_v, zer_v, deg_s, sem):
        c = lax.axis_index("c")
        s = lax.axis_index("s")
        w = c * NS + s
        for i in range(ROWS_PER_TILE // 16):
            zer_v[pl.ds(i * 16, 16)] = jnp.zeros((16,), jnp.float32)
        for i in range(B // 16):
            ones_v[pl.ds(i * 16, 16)] = jnp.ones((16,), jnp.float32)
        pltpu.sync_copy(zer_v, deg_s.at[pl.ds(s * ROWS_PER_TILE, ROWS_PER_TILE)])
        pltpu.sync_copy(dst_hbm.at[w], idx_v)
        plsc.subcore_barrier()

        def issue(j, carry):
            pltpu.async_copy(ones_v, deg_s.at[idx_v.at[j]], sem, add=True)
            return carry

        lax.fori_loop(0, CH, issue, 0)

        def drain(j, carry):
            pltpu.make_async_copy(ones_v, deg_s.at[idx_v.at[j]], sem).wait()
            return carry

        lax.fori_loop(0, CH, drain, 0)
        plsc.subcore_barrier()

        @pl.when(s == 0)
        def _():
            pltpu.sync_copy(deg_s, out_hbm.at[pl.ds(c * NT, NT)])

    return hist


@functools.lru_cache(maxsize=None)
def _build_agg():
    @functools.partial(
        pl.kernel,
        mesh=_mesh(),
        out_type=jax.ShapeDtypeStruct((NC * NT, D), jnp.float32),
        scratch_types=[
            pltpu.VMEM((HCH, B), jnp.int32),
            pltpu.VMEM((HCH, B), jnp.int32),
            pltpu.VMEM((B, D), jnp.float32),
            pltpu.VMEM((B, D), jnp.float32),
            pltpu.VMEM_SHARED((NT, D), jnp.float32),
            pltpu.SemaphoreType.DMA,
            pltpu.SemaphoreType.DMA,
        ],
    )
    def agg(dst_hbm, src_hbm, z_hbm, out_hbm,
            dst_v, src_v, buf0, buf1, acc_s, g0a, g1a):
        c = lax.axis_index("c")
        s = lax.axis_index("s")
        w = c * NS + s
        r0 = s * ROWS_PER_TILE

        def zrow(r, carry):
            for i in range(D // 16):
                buf0[r, pl.ds(i * 16, 16)] = jnp.zeros((16,), jnp.float32)
            return carry

        lax.fori_loop(0, B, zrow, 0)
        for r in range(ROWS_PER_TILE // B):
            pltpu.sync_copy(buf0, acc_s.at[pl.ds(r0 + r * B, B)])
        plsc.subcore_barrier()

        def gather(j, buf, sem):
            pltpu.async_copy(z_hbm.at[dst_v.at[j]], buf, sem)

        def gather_wait(j, buf, sem):
            pltpu.make_async_copy(z_hbm.at[dst_v.at[j]], buf, sem).wait()

        def scatter(j, buf):
            pltpu.sync_copy(buf, acc_s.at[src_v.at[j]], add=True)

        for h in range(2):
            pltpu.sync_copy(dst_hbm.at[w, pl.ds(h * HCH, HCH)], dst_v)
            pltpu.sync_copy(src_hbm.at[w, pl.ds(h * HCH, HCH)], src_v)
            gather(0, buf0, g0a)

            def body(i, carry):
                j0 = i * 2
                j1 = j0 + 1
                gather(j1, buf1, g1a)
                gather_wait(j0, buf0, g0a)
                scatter(j0, buf0)

                @pl.when(j1 + 1 < HCH)
                def _():
                    gather(j1 + 1, buf0, g0a)

                gather_wait(j1, buf1, g1a)
                scatter(j1, buf1)
                return carry

            lax.fori_loop(0, HCH // 2, body, 0)

        plsc.subcore_barrier()
        pltpu.sync_copy(
            acc_s.at[pl.ds(r0, ROWS_PER_TILE)],
            out_hbm.at[pl.ds(c * NT + r0, ROWS_PER_TILE)],
        )

    return agg


def _matmul_kernel(x_ref, w_ref, y_ref):
    y_ref[...] = lax.dot_general(
        x_ref[...], w_ref[...], (((1,), (1,)), ((), ())),
        preferred_element_type=jnp.float32,
    )


def _scale_kernel(y_ref, deg_ref, z_ref):
    i = pl.program_id(0)
    deg = deg_ref[0] + deg_ref[1]
    row = lax.broadcasted_iota(jnp.int32, deg.shape, 0) + i * deg.shape[0]
    dinv = jnp.where(row < N, lax.rsqrt(deg), 0.0)
    z_ref[...] = dinv * y_ref[...]


def _finish_kernel(acc_ref, deg_ref, b_ref, o_ref):
    acc = acc_ref[0] + acc_ref[1]
    dinv = lax.rsqrt(deg_ref[0] + deg_ref[1])
    o_ref[...] = dinv * acc + b_ref[...]


def kernel(x, edge_index, W, b):
    src = edge_index[0]
    dst = edge_index[1]
    pad = N + (jnp.arange(EP - E, dtype=jnp.int32) % (NT - N))
    dsts = jnp.concatenate([dst, pad]).reshape(NW, CH, B)
    srcs = jnp.concatenate([src, pad]).reshape(NW, CH, B)

    deg = _build_hist()(dsts)
    deg2 = deg.reshape(NC, NT, 1)

    blk = 640
    y = pl.pallas_call(
        _matmul_kernel,
        grid=(NT // blk,),
        in_specs=[
            pl.BlockSpec((blk, D), lambda i: (i, 0)),
            pl.BlockSpec((D, D), lambda i: (0, 0)),
        ],
        out_specs=pl.BlockSpec((blk, D), lambda i: (i, 0)),
        out_shape=jax.ShapeDtypeStruct((NT, D), jnp.float32),
    )(x, W)
    z = pl.pallas_call(
        _scale_kernel,
        grid=(NT // blk,),
        in_specs=[
            pl.BlockSpec((blk, D), lambda i: (i, 0)),
            pl.BlockSpec((NC, blk, 1), lambda i: (0, i, 0)),
        ],
        out_specs=pl.BlockSpec((blk, D), lambda i: (i, 0)),
        out_shape=jax.ShapeDtypeStruct((NT, D), jnp.float32),
    )(y, deg2)

    acc = _build_agg()(dsts, srcs, z)

    oblk = 2000
    out = pl.pallas_call(
        _finish_kernel,
        grid=(N // oblk,),
        in_specs=[
            pl.BlockSpec((NC, oblk, D), lambda i: (0, i, 0)),
            pl.BlockSpec((NC, oblk, 1), lambda i: (0, i, 0)),
            pl.BlockSpec((1, D), lambda i: (0, 0)),
        ],
        out_specs=pl.BlockSpec((oblk, D), lambda i: (i, 0)),
        out_shape=jax.ShapeDtypeStruct((N, D), jnp.float32),
    )(acc.reshape(NC, NT, D), deg2, b.reshape(1, D))
    return out


# speedup vs baseline: 1.1064x; 1.1064x over previous
import functools

import jax
import jax.numpy as jnp
from jax import lax
from jax.experimental import pallas as pl
from jax.experimental.pallas import tpu as pltpu
from jax.experimental.pallas import tpu_sc as plsc

N = 10000
D = 128
E = 320000
NT = 10240
NC = 2
NS = 16
NW = NC * NS
B = 128
CH = 80
MAIN = 72
H0 = 40
H1 = CH - H0
EROWS = 2500
TAILE = E - NW * MAIN * B
TPAD = NW * 8 * B - TAILE
ROWS_PER_TILE = NT // NS


def _mesh():
    return plsc.VectorSubcoreMesh(
        core_axis_name="c", subcore_axis_name="s", num_cores=NC, num_subcores=NS
    )


@functools.lru_cache(maxsize=None)
def _build_hist():
    @functools.partial(
        pl.kernel,
        mesh=_mesh(),
        out_type=jax.ShapeDtypeStruct((NC, NT * 8), jnp.float32),
        scratch_types=[
            pltpu.VMEM((CH, B), jnp.int32),
            pltpu.VMEM((B,), jnp.float32),
            pltpu.VMEM((ROWS_PER_TILE,), jnp.float32),
            pltpu.VMEM_SHARED((NT * 8,), jnp.float32),
            pltpu.SemaphoreType.DMA,
        ],
    )
    def hist(ei_hbm, tail_hbm, out_hbm, idx_v, ones_v, zer_v, deg_s, sem):
        c = lax.axis_index("c")
        s = lax.axis_index("s")
        w = c * NS + s
        for i in range(ROWS_PER_TILE // 16):
            zer_v[pl.ds(i * 16, 16)] = jnp.zeros((16,), jnp.float32)
        for i in range(B // 16):
            ones_v[pl.ds(i * 16, 16)] = jnp.ones((16,), jnp.float32)
        for k in range(8):
            pltpu.sync_copy(
                zer_v,
                deg_s.at[pl.ds(s * (ROWS_PER_TILE * 8) + k * ROWS_PER_TILE,
                               ROWS_PER_TILE)],
            )
        pltpu.sync_copy(ei_hbm.at[1, pl.ds(w * MAIN, MAIN)],
                        idx_v.at[pl.ds(0, MAIN)])
        pltpu.sync_copy(tail_hbm.at[1, w], idx_v.at[pl.ds(MAIN, 8)])

        def scale8(j, carry):
            for i in range(B // 16):
                v = idx_v[j, pl.ds(i * 16, 16)]
                idx_v[j, pl.ds(i * 16, 16)] = v * 8
            return carry

        lax.fori_loop(0, CH, scale8, 0)
        plsc.subcore_barrier()

        def issue(j, carry):
            pltpu.async_copy(ones_v, deg_s.at[idx_v.at[j]], sem, add=True)
            return carry

        lax.fori_loop(0, CH, issue, 0)

        def drain(j, carry):
            pltpu.make_async_copy(ones_v, deg_s.at[idx_v.at[j]], sem).wait()
            return carry

        lax.fori_loop(0, CH, drain, 0)
        plsc.subcore_barrier()

        @pl.when(s == 0)
        def _():
            pltpu.sync_copy(deg_s, out_hbm.at[c])

    return hist


@functools.lru_cache(maxsize=None)
def _build_agg():
    @functools.partial(
        pl.kernel,
        mesh=_mesh(),
        out_type=jax.ShapeDtypeStruct((NC * NT, D), jnp.float32),
        scratch_types=[
            pltpu.VMEM((H0, B), jnp.int32),
            pltpu.VMEM((H0, B), jnp.int32),
            pltpu.VMEM((B, D), jnp.float32),
            pltpu.VMEM((B, D), jnp.float32),
            pltpu.VMEM_SHARED((NT, D), jnp.float32),
            pltpu.SemaphoreType.DMA,
            pltpu.SemaphoreType.DMA,
        ],
    )
    def agg(ei_hbm, tail_hbm, z_hbm, out_hbm,
            dst_v, src_v, buf0, buf1, acc_s, g0a, g1a):
        c = lax.axis_index("c")
        s = lax.axis_index("s")
        w = c * NS + s
        r0 = s * ROWS_PER_TILE

        def zrow(r, carry):
            for i in range(D // 16):
                buf0[r, pl.ds(i * 16, 16)] = jnp.zeros((16,), jnp.float32)
            return carry

        lax.fori_loop(0, B, zrow, 0)
        for r in range(ROWS_PER_TILE // B):
            pltpu.sync_copy(buf0, acc_s.at[pl.ds(r0 + r * B, B)])
        plsc.subcore_barrier()

        def gather(j, buf, sem):
            pltpu.async_copy(z_hbm.at[dst_v.at[j]], buf, sem)

        def gather_wait(j, buf, sem):
            pltpu.make_async_copy(z_hbm.at[dst_v.at[j]], buf, sem).wait()

        def scatter(j, buf):
            pltpu.sync_copy(buf, acc_s.at[src_v.at[j]], add=True)

        def run_half(nch):
            gather(0, buf0, g0a)

            def body(i, carry):
                j0 = i * 2
                j1 = j0 + 1
                gather(j1, buf1, g1a)
                gather_wait(j0, buf0, g0a)
                scatter(j0, buf0)

                @pl.when(j1 + 1 < nch)
                def _():
                    gather(j1 + 1, buf0, g0a)

                gather_wait(j1, buf1, g1a)
                scatter(j1, buf1)
                return carry

            lax.fori_loop(0, nch // 2, body, 0)

        pltpu.sync_copy(ei_hbm.at[1, pl.ds(w * MAIN, H0)], dst_v)
        pltpu.sync_copy(ei_hbm.at[0, pl.ds(w * MAIN, H0)], src_v)
        run_half(H0)
        pltpu.sync_copy(ei_hbm.at[1, pl.ds(w * MAIN + H0, MAIN - H0)],
                        dst_v.at[pl.ds(0, MAIN - H0)])
        pltpu.sync_copy(ei_hbm.at[0, pl.ds(w * MAIN + H0, MAIN - H0)],
                        src_v.at[pl.ds(0, MAIN - H0)])
        pltpu.sync_copy(tail_hbm.at[1, w], dst_v.at[pl.ds(MAIN - H0, 8)])
        pltpu.sync_copy(tail_hbm.at[0, w], src_v.at[pl.ds(MAIN - H0, 8)])
        run_half(H1)

        plsc.subcore_barrier()
        pltpu.sync_copy(
            acc_s.at[pl.ds(r0, ROWS_PER_TILE)],
            out_hbm.at[pl.ds(c * NT + r0, ROWS_PER_TILE)],
        )

    return agg


def _scale_matmul_kernel(x_ref, w_ref, deg_ref, z_ref):
    i = pl.program_id(0)
    y = lax.dot_general(
        x_ref[...], w_ref[...], (((1,), (1,)), ((), ())),
        preferred_element_type=jnp.float32,
    )
    deg = deg_ref[0, :, 0:1] + deg_ref[1, :, 0:1]
    row = lax.broadcasted_iota(jnp.int32, deg.shape, 0) + i * deg.shape[0]
    dinv = jnp.where(row < N, lax.rsqrt(deg), 0.0)
    z_ref[...] = dinv * y


def _finish_kernel(acc_ref, deg_ref, b_ref, o_ref):
    acc = acc_ref[0] + acc_ref[1]
    dinv = lax.rsqrt(deg_ref[0, :, 0:1] + deg_ref[1, :, 0:1])
    o_ref[...] = dinv * acc + b_ref[...]


def kernel(x, edge_index, W, b):
    ei = edge_index.reshape(2, EROWS, B)
    padv = N + (jnp.arange(TPAD, dtype=jnp.int32) % (NT - N))
    tail = jnp.concatenate(
        [edge_index[:, NW * MAIN * B:], jnp.broadcast_to(padv, (2, TPAD))],
        axis=1,
    ).reshape(2, NW, 8, B)

    deg = _build_hist()(ei, tail)
    deg2 = deg.reshape(NC, NT, 8)

    blk = 1280
    z = pl.pallas_call(
        _scale_matmul_kernel,
        grid=(NT // blk,),
        in_specs=[
            pl.BlockSpec((blk, D), lambda i: (i, 0)),
            pl.BlockSpec((D, D), lambda i: (0, 0)),
            pl.BlockSpec((NC, blk, 8), lambda i: (0, i, 0)),
        ],
        out_specs=pl.BlockSpec((blk, D), lambda i: (i, 0)),
        out_shape=jax.ShapeDtypeStruct((NT, D), jnp.float32),
    )(x, W, deg2)

    acc = _build_agg()(ei, tail, z)

    oblk = 2048
    out = pl.pallas_call(
        _finish_kernel,
        grid=(NT // oblk,),
        in_specs=[
            pl.BlockSpec((NC, oblk, D), lambda i: (0, i, 0)),
            pl.BlockSpec((NC, oblk, 8), lambda i: (0, i, 0)),
            pl.BlockSpec((1, D), lambda i: (0, 0)),
        ],
        out_specs=pl.BlockSpec((oblk, D), lambda i: (i, 0)),
        out_shape=jax.ShapeDtypeStruct((N, D), jnp.float32),
    )(acc.reshape(NC, NT, D), deg2, b.reshape(1, D))
    return out


# speedup vs baseline: 1.2143x; 1.0975x over previous
import functools

import jax
import jax.numpy as jnp
from jax import lax
from jax.experimental import pallas as pl
from jax.experimental.pallas import tpu as pltpu
from jax.experimental.pallas import tpu_sc as plsc

N = 10000
D = 128
E = 320000
NT = 10240
NC = 2
NS = 16
NW = NC * NS
B = 128
CH = 80
MAIN = 72
H0 = 40
H1 = CH - H0
EROWS = 2500
TAILE = E - NW * MAIN * B
TPAD = NW * 8 * B - TAILE
ROWS_PER_TILE = NT // NS


def _mesh():
    return plsc.VectorSubcoreMesh(
        core_axis_name="c", subcore_axis_name="s", num_cores=NC, num_subcores=NS
    )


@functools.lru_cache(maxsize=None)
def _build_hist():
    @functools.partial(
        pl.kernel,
        mesh=_mesh(),
        out_type=jax.ShapeDtypeStruct((NC, NT), jnp.float32),
        scratch_types=[
            pltpu.VMEM((CH, B), jnp.int32),
            pltpu.VMEM((B,), jnp.float32),
            pltpu.VMEM((ROWS_PER_TILE,), jnp.float32),
            pltpu.VMEM_SHARED((NT,), jnp.float32),
            pltpu.SemaphoreType.DMA,
        ],
    )
    def hist(ei_hbm, tail_hbm, out_hbm, idx_v, ones_v, zer_v, deg_s, sem):
        c = lax.axis_index("c")
        s = lax.axis_index("s")
        w = c * NS + s
        for i in range(ROWS_PER_TILE // 16):
            zer_v[pl.ds(i * 16, 16)] = jnp.zeros((16,), jnp.float32)
        for i in range(B // 16):
            ones_v[pl.ds(i * 16, 16)] = jnp.ones((16,), jnp.float32)
        pltpu.sync_copy(zer_v, deg_s.at[pl.ds(s * ROWS_PER_TILE, ROWS_PER_TILE)])
        pltpu.sync_copy(ei_hbm.at[1, pl.ds(w * MAIN, MAIN)],
                        idx_v.at[pl.ds(0, MAIN)])
        pltpu.sync_copy(tail_hbm.at[1, w], idx_v.at[pl.ds(MAIN, 8)])
        plsc.subcore_barrier()

        def issue(j, carry):
            pltpu.async_copy(ones_v, deg_s.at[idx_v.at[j]], sem, add=True)
            return carry

        lax.fori_loop(0, CH, issue, 0)

        def drain(j, carry):
            pltpu.make_async_copy(ones_v, deg_s.at[idx_v.at[j]], sem).wait()
            return carry

        lax.fori_loop(0, CH, drain, 0)
        plsc.subcore_barrier()

        @pl.when(s == 0)
        def _():
            pltpu.sync_copy(deg_s, out_hbm.at[c])

    return hist


@functools.lru_cache(maxsize=None)
def _build_agg():
    @functools.partial(
        pl.kernel,
        mesh=_mesh(),
        out_type=jax.ShapeDtypeStruct((NC * NT, D), jnp.float32),
        scratch_types=[
            pltpu.VMEM((H0, B), jnp.int32),
            pltpu.VMEM((H0, B), jnp.int32),
            pltpu.VMEM((B, D), jnp.float32),
            pltpu.VMEM((B, D), jnp.float32),
            pltpu.VMEM_SHARED((NT, D), jnp.float32),
            pltpu.SemaphoreType.DMA,
            pltpu.SemaphoreType.DMA,
        ],
    )
    def agg(ei_hbm, tail_hbm, z_hbm, out_hbm,
            dst_v, src_v, buf0, buf1, acc_s, g0a, g1a):
        c = lax.axis_index("c")
        s = lax.axis_index("s")
        w = c * NS + s
        r0 = s * ROWS_PER_TILE

        def zrow(r, carry):
            for i in range(D // 16):
                buf0[r, pl.ds(i * 16, 16)] = jnp.zeros((16,), jnp.float32)
            return carry

        lax.fori_loop(0, B, zrow, 0)
        for r in range(ROWS_PER_TILE // B):
            pltpu.sync_copy(buf0, acc_s.at[pl.ds(r0 + r * B, B)])
        plsc.subcore_barrier()

        def gather(j, buf, sem):
            pltpu.async_copy(z_hbm.at[dst_v.at[j]], buf, sem)

        def gather_wait(j, buf, sem):
            pltpu.make_async_copy(z_hbm.at[dst_v.at[j]], buf, sem).wait()

        def scatter(j, buf):
            pltpu.sync_copy(buf, acc_s.at[src_v.at[j]], add=True)

        def run_half(nch):
            gather(0, buf0, g0a)

            def body(i, carry):
                j0 = i * 2
                j1 = j0 + 1
                gather(j1, buf1, g1a)
                gather_wait(j0, buf0, g0a)
                scatter(j0, buf0)

                @pl.when(j1 + 1 < nch)
                def _():
                    gather(j1 + 1, buf0, g0a)

                gather_wait(j1, buf1, g1a)
                scatter(j1, buf1)
                return carry

            lax.fori_loop(0, nch // 2, body, 0)

        pltpu.sync_copy(ei_hbm.at[1, pl.ds(w * MAIN, H0)], dst_v)
        pltpu.sync_copy(ei_hbm.at[0, pl.ds(w * MAIN, H0)], src_v)
        run_half(H0)
        pltpu.sync_copy(ei_hbm.at[1, pl.ds(w * MAIN + H0, MAIN - H0)],
                        dst_v.at[pl.ds(0, MAIN - H0)])
        pltpu.sync_copy(ei_hbm.at[0, pl.ds(w * MAIN + H0, MAIN - H0)],
                        src_v.at[pl.ds(0, MAIN - H0)])
        pltpu.sync_copy(tail_hbm.at[1, w], dst_v.at[pl.ds(MAIN - H0, 8)])
        pltpu.sync_copy(tail_hbm.at[0, w], src_v.at[pl.ds(MAIN - H0, 8)])
        run_half(H1)

        plsc.subcore_barrier()
        pltpu.sync_copy(
            acc_s.at[pl.ds(r0, ROWS_PER_TILE)],
            out_hbm.at[pl.ds(c * NT + r0, ROWS_PER_TILE)],
        )

    return agg


def _diag_scale(dinv_row):
    r = lax.broadcasted_iota(jnp.int32, (D, D), 0)
    l = lax.broadcasted_iota(jnp.int32, (D, D), 1)
    return jnp.where(r == l, jnp.broadcast_to(dinv_row, (D, D)), 0.0)


def _scale_matmul_kernel(x_ref, w_ref, deg_ref, z_ref):
    i = pl.program_id(0)
    y = lax.dot_general(
        x_ref[...], w_ref[...], (((1,), (1,)), ((), ())),
        preferred_element_type=jnp.float32,
    )
    nsub = y.shape[0] // D
    for k in range(nsub):
        row = i * nsub + k
        dvec = deg_ref[0, pl.ds(row, 1), :] + deg_ref[1, pl.ds(row, 1), :]
        grow = row * D + lax.broadcasted_iota(jnp.int32, (1, D), 1)
        dinv = jnp.where(grow < N, lax.rsqrt(dvec), 0.0)
        z_ref[pl.ds(k * D, D), :] = lax.dot_general(
            _diag_scale(dinv), y[k * D:(k + 1) * D, :],
            (((1,), (0,)), ((), ())), preferred_element_type=jnp.float32,
        )


def _finish_kernel(acc_ref, deg_ref, b_ref, o_ref):
    i = pl.program_id(0)
    acc = acc_ref[0] + acc_ref[1]
    nsub = acc.shape[0] // D
    for k in range(nsub):
        row = i * nsub + k
        dvec = deg_ref[0, pl.ds(row, 1), :] + deg_ref[1, pl.ds(row, 1), :]
        dinv = lax.rsqrt(dvec)
        o_ref[pl.ds(k * D, D), :] = lax.dot_general(
            _diag_scale(dinv), acc[k * D:(k + 1) * D, :],
            (((1,), (0,)), ((), ())), preferred_element_type=jnp.float32,
        ) + b_ref[...]


def kernel(x, edge_index, W, b):
    ei = edge_index.reshape(2, EROWS, B)
    padv = N + (jnp.arange(TPAD, dtype=jnp.int32) % (NT - N))
    tail = jnp.concatenate(
        [edge_index[:, NW * MAIN * B:], jnp.broadcast_to(padv, (2, TPAD))],
        axis=1,
    ).reshape(2, NW, 8, B)

    deg = _build_hist()(ei, tail)
    deg2 = deg.reshape(NC, NT // D, D)

    blk = 1280
    z = pl.pallas_call(
        _scale_matmul_kernel,
        grid=(NT // blk,),
        in_specs=[
            pl.BlockSpec((blk, D), lambda i: (i, 0)),
            pl.BlockSpec((D, D), lambda i: (0, 0)),
            pl.BlockSpec((NC, NT // D, D), lambda i: (0, 0, 0)),
        ],
        out_specs=pl.BlockSpec((blk, D), lambda i: (i, 0)),
        out_shape=jax.ShapeDtypeStruct((NT, D), jnp.float32),
    )(x, W, deg2)

    acc = _build_agg()(ei, tail, z)

    oblk = 2048
    out = pl.pallas_call(
        _finish_kernel,
        grid=(NT // oblk,),
        in_specs=[
            pl.BlockSpec((NC, oblk, D), lambda i: (0, i, 0)),
            pl.BlockSpec((NC, NT // D, D), lambda i: (0, 0, 0)),
            pl.BlockSpec((1, D), lambda i: (0, 0)),
        ],
        out_specs=pl.BlockSpec((oblk, D), lambda i: (i, 0)),
        out_shape=jax.ShapeDtypeStruct((N, D), jnp.float32),
    )(acc.reshape(NC, NT, D), deg2, b.reshape(1, D))
    return out
